# baseline (speedup 1.0000x reference)
import functools
import math

import jax
import jax.numpy as jnp
from jax import lax
from jax.experimental import pallas as pl
from jax.experimental.pallas import tpu as pltpu
from jax.experimental.pallas import tpu_sc as plsc

_C = 100000
_B = 1024
_SMOOTHING = 0.1
_CONF = 1.0 - _SMOOTHING
_S = _SMOOTHING / (_C - 1)

_CW = 512
_NCH = 132
_SW = _NCH * _CW
_W = 2048
_TC0 = _SW // _W
_NBLK = (_C - _SW + _W - 1) // _W

_NW = 32
_RPW = _B // _NW
_NEG_INF = float("-inf")


def _tc_stream_kernel(tgt_ref, x_ref, m_out, se_out, rs_out, g_out,
                      m_ref, se_ref, rs_ref, g_ref):
    i = pl.program_id(0)

    @pl.when(i == 0)
    def _init():
        m_ref[...] = jnp.full_like(m_ref, _NEG_INF)
        se_ref[...] = jnp.zeros_like(se_ref)
        rs_ref[...] = jnp.zeros_like(rs_ref)
        g_ref[...] = jnp.zeros_like(g_ref)

    x = x_ref[...]
    m = m_ref[...]
    col = jax.lax.broadcasted_iota(jnp.int32, x.shape, 1) + (_TC0 + i) * _W
    g_ref[...] += jnp.sum(
        jnp.where(col == tgt_ref[...], x, 0.0), axis=1, keepdims=True
    )

    @pl.when(i < _NBLK - 1)
    def _main():
        bm = jnp.max(x, axis=1, keepdims=True)
        nm = jnp.maximum(m, bm)
        se_ref[...] = se_ref[...] * jnp.exp(m - nm) + jnp.sum(
            jnp.exp(x - nm), axis=1, keepdims=True
        )
        m_ref[...] = nm
        rs_ref[...] += jnp.sum(x, axis=1, keepdims=True)

    @pl.when(i == _NBLK - 1)
    def _last():
        valid = col < _C
        xm = jnp.where(valid, x, _NEG_INF)
        bm = jnp.max(xm, axis=1, keepdims=True)
        nm = jnp.maximum(m, bm)
        se_ref[...] = se_ref[...] * jnp.exp(m - nm) + jnp.sum(
            jnp.exp(xm - nm), axis=1, keepdims=True
        )
        rs_ref[...] += jnp.sum(jnp.where(valid, x, 0.0), axis=1, keepdims=True)
        m_out[...] = nm
        se_out[...] = se_ref[...]
        rs_out[...] = rs_ref[...]
        g_out[...] = g_ref[...]


def _tc_stream(pred, tgt):
    return pl.pallas_call(
        _tc_stream_kernel,
        grid=(_NBLK,),
        in_specs=[
            pl.BlockSpec((_B, 1), lambda i: (0, 0)),
            pl.BlockSpec((_B, _W), lambda i: (0, _TC0 + i)),
        ],
        out_specs=[
            pl.BlockSpec((_B, 1), lambda i: (0, 0)),
            pl.BlockSpec((_B, 1), lambda i: (0, 0)),
            pl.BlockSpec((_B, 1), lambda i: (0, 0)),
            pl.BlockSpec((_B, 1), lambda i: (0, 0)),
        ],
        out_shape=[jax.ShapeDtypeStruct((_B, 1), jnp.float32)] * 4,
        scratch_shapes=[pltpu.VMEM((_B, 1), jnp.float32)] * 4,
        compiler_params=pltpu.CompilerParams(
            dimension_semantics=("arbitrary",),
        ),
    )(tgt, pred)


def _sc_kernel(pred2d, tgt, m_hbm, se_hbm, rs_hbm, g_hbm,
               tgt_v, buf, macc, sacc, rsacc, m_v, se_v, rs_v, g_v,
               sem0, sem1):
    wid = lax.axis_index("s") * 2 + lax.axis_index("c")
    base = wid * _RPW
    lanes = lax.broadcasted_iota(jnp.int32, (16,), 0)

    pltpu.sync_copy(tgt.at[pl.ds(base, _RPW)], tgt_v)
    t0 = tgt_v[pl.ds(0, 16)]
    t1 = tgt_v[pl.ds(16, 16)]

    for r in range(_RPW):
        macc[r, :] = jnp.full((16,), _NEG_INF, jnp.float32)
        sacc[r, :] = jnp.zeros((16,), jnp.float32)
        rsacc[r, :] = jnp.zeros((16,), jnp.float32)

    def _start(ch, b):
        return pltpu.async_copy(
            pred2d.at[pl.ds(base, _RPW), pl.ds(ch * _CW, _CW)],
            buf.at[b],
            sem0 if b == 0 else sem1,
        )

    def _wait(b):
        pltpu.make_async_copy(
            pred2d.at[pl.ds(0, _RPW), pl.ds(0, _CW)],
            buf.at[b],
            sem0 if b == 0 else sem1,
        ).wait()

    def _consume(b, ch, g0, g1):
        for r in range(_RPW):
            def _vbody(j, carry, b=b, r=r):
                mv, sv, rv = carry
                x = buf.at[b, r][pl.ds(j, 16)]
                nm = jnp.maximum(mv, x)
                sv = sv * jnp.exp(mv - nm) + jnp.exp(x - nm)
                rv = rv + x
                return nm, sv, rv

            mv, sv, rv = plsc.parallel_loop(
                0, _CW, step=16, unroll=8,
                carry=(macc[r, :], sacc[r, :], rsacc[r, :]),
            )(_vbody)
            macc[r, :] = mv
            sacc[r, :] = sv
            rsacc[r, :] = rv

        lo = ch * _CW
        out = []
        for k, tk in ((0, g0), (1, g1)):
            c_in = (t0 if k == 0 else t1) - lo
            hit = (c_in >= 0) & (c_in < _CW)
            c_cl = jnp.clip(c_in, 0, _CW - 1)
            val = plsc.load_gather(buf.at[b], [k * 16 + lanes, c_cl])
            out.append(jnp.where(hit, val, tk))
        return out[0], out[1]

    _start(0, 0)
    _start(1, 1)

    def _pair(i, carry):
        g0, g1 = carry
        ch0 = 2 * i
        _wait(0)
        g0, g1 = _consume(0, ch0, g0, g1)

        @pl.when(ch0 + 2 < _NCH)
        def _s0():
            _start(ch0 + 2, 0)

        _wait(1)
        g0, g1 = _consume(1, ch0 + 1, g0, g1)

        @pl.when(ch0 + 3 < _NCH)
        def _s1():
            _start(ch0 + 3, 1)

        return g0, g1

    zero16 = jnp.zeros((16,), jnp.float32)
    g0, g1 = lax.fori_loop(0, _NCH // 2, _pair, (zero16, zero16))
    g_v[pl.ds(0, 16)] = g0
    g_v[pl.ds(16, 16)] = g1

    mrow_vec = zero16
    serow_vec = zero16
    rsrow_vec = zero16
    for r in range(_RPW):
        mv = macc[r, :]
        sv = sacc[r, :]
        rv = rsacc[r, :]
        mrow = jnp.max(mv)
        serow = jnp.sum(sv * jnp.exp(mv - jnp.broadcast_to(mrow, (16,))))
        rsrow = jnp.sum(rv)
        hot = lanes == (r % 16)
        mrow_vec = jnp.where(hot, jnp.broadcast_to(mrow, (16,)), mrow_vec)
        serow_vec = jnp.where(hot, jnp.broadcast_to(serow, (16,)), serow_vec)
        rsrow_vec = jnp.where(hot, jnp.broadcast_to(rsrow, (16,)), rsrow_vec)
        if r % 16 == 15:
            o = (r // 16) * 16
            m_v[pl.ds(o, 16)] = mrow_vec
            se_v[pl.ds(o, 16)] = serow_vec
            rs_v[pl.ds(o, 16)] = rsrow_vec
    pltpu.sync_copy(m_v, m_hbm.at[pl.ds(base, _RPW)])
    pltpu.sync_copy(se_v, se_hbm.at[pl.ds(base, _RPW)])
    pltpu.sync_copy(rs_v, rs_hbm.at[pl.ds(base, _RPW)])
    pltpu.sync_copy(g_v, g_hbm.at[pl.ds(base, _RPW)])


def _sc_stream(pred, tgt):
    mesh = plsc.VectorSubcoreMesh(core_axis_name="c", subcore_axis_name="s")
    f = functools.partial(
        pl.kernel,
        out_type=[jax.ShapeDtypeStruct((_B,), jnp.float32)] * 4,
        mesh=mesh,
        scratch_types=[
            pltpu.VMEM((_RPW,), jnp.int32),
            pltpu.VMEM((2, _RPW, _CW), jnp.float32),
            pltpu.VMEM((_RPW, 16), jnp.float32),
            pltpu.VMEM((_RPW, 16), jnp.float32),
            pltpu.VMEM((_RPW, 16), jnp.float32),
            pltpu.VMEM((_RPW,), jnp.float32),
            pltpu.VMEM((_RPW,), jnp.float32),
            pltpu.VMEM((_RPW,), jnp.float32),
            pltpu.VMEM((_RPW,), jnp.float32),
            pltpu.SemaphoreType.DMA,
            pltpu.SemaphoreType.DMA,
        ],
        compiler_params=pltpu.CompilerParams(
            use_tc_tiling_on_sc=False, needs_layout_passes=False
        ),
    )(_sc_kernel)
    return f(pred, tgt)


def _merge_kernel(m1_ref, se1_ref, rs1_ref, g1_ref, m2_ref, se2_ref,
                  rs2_ref, g2_ref, out_ref):
    m1 = m1_ref[...]
    m2 = m2_ref[...]
    nm = jnp.maximum(m1, m2)
    se = se1_ref[...] * jnp.exp(m1 - nm) + se2_ref[...] * jnp.exp(m2 - nm)
    z = nm + jnp.log(se)
    zsum = jnp.sum(z)
    rssum = jnp.sum(rs1_ref[...]) + jnp.sum(rs2_ref[...])
    gsum = jnp.sum(g1_ref[...]) + jnp.sum(g2_ref[...])
    kc = _SMOOTHING * math.log(_S) + _CONF * math.log(_CONF)
    total = (
        _B * kc
        - _S * (rssum - _C * zsum)
        - (_CONF - _S) * (gsum - zsum)
    )
    out_ref[0, 0] = total / (_B * _C)


def _merge(m1, se1, rs1, g1, m2, se2, rs2, g2):
    return pl.pallas_call(
        _merge_kernel,
        in_specs=[pl.BlockSpec((_B, 1), lambda: (0, 0))] * 8,
        out_specs=pl.BlockSpec((1, 1), lambda: (0, 0), memory_space=pltpu.SMEM),
        out_shape=jax.ShapeDtypeStruct((1, 1), jnp.float32),
    )(m1, se1, rs1, g1, m2, se2, rs2, g2)


def kernel(pred, target):
    tgt = target.astype(jnp.int32)
    m2, se2, rs2, g2 = _sc_stream(pred, tgt)
    m1, se1, rs1, g1 = _tc_stream(pred, tgt.reshape(_B, 1))
    out = _merge(
        m1, se1, rs1, g1,
        m2.reshape(_B, 1), se2.reshape(_B, 1), rs2.reshape(_B, 1),
        g2.reshape(_B, 1),
    )
    return out[0, 0]


# speedup vs baseline: 1.3416x; 1.3416x over previous
import functools
import math

import jax
import jax.numpy as jnp
from jax import lax
from jax.experimental import pallas as pl
from jax.experimental.pallas import tpu as pltpu
from jax.experimental.pallas import tpu_sc as plsc

_C = 100000
_B = 1024
_SMOOTHING = 0.1
_CONF = 1.0 - _SMOOTHING
_S = _SMOOTHING / (_C - 1)

_CW = 1024
_NCH = 66
_SW = _NCH * _CW
_W = 2048
_TC0 = _SW // _W
_NBLK = (_C - _SW + _W - 1) // _W

_NW = 32
_RPW = _B // _NW
_NEG_INF = float("-inf")


def _tc_stream_kernel(tgt_ref, x_ref, m_out, se_out, rs_out, g_out,
                      m_ref, se_ref, rs_ref, g_ref):
    i = pl.program_id(0)

    @pl.when(i == 0)
    def _init():
        m_ref[...] = jnp.full_like(m_ref, _NEG_INF)
        se_ref[...] = jnp.zeros_like(se_ref)
        rs_ref[...] = jnp.zeros_like(rs_ref)
        g_ref[...] = jnp.zeros_like(g_ref)

    x = x_ref[...]
    m = m_ref[...]
    col = jax.lax.broadcasted_iota(jnp.int32, x.shape, 1) + (_TC0 + i) * _W
    g_ref[...] += jnp.sum(
        jnp.where(col == tgt_ref[...], x, 0.0), axis=1, keepdims=True
    )

    @pl.when(i < _NBLK - 1)
    def _main():
        bm = jnp.max(x, axis=1, keepdims=True)
        nm = jnp.maximum(m, bm)
        se_ref[...] = se_ref[...] * jnp.exp(m - nm) + jnp.sum(
            jnp.exp(x - nm), axis=1, keepdims=True
        )
        m_ref[...] = nm
        rs_ref[...] += jnp.sum(x, axis=1, keepdims=True)

    @pl.when(i == _NBLK - 1)
    def _last():
        valid = col < _C
        xm = jnp.where(valid, x, _NEG_INF)
        bm = jnp.max(xm, axis=1, keepdims=True)
        nm = jnp.maximum(m, bm)
        se_ref[...] = se_ref[...] * jnp.exp(m - nm) + jnp.sum(
            jnp.exp(xm - nm), axis=1, keepdims=True
        )
        rs_ref[...] += jnp.sum(jnp.where(valid, x, 0.0), axis=1, keepdims=True)
        m_out[...] = nm
        se_out[...] = se_ref[...]
        rs_out[...] = rs_ref[...]
        g_out[...] = g_ref[...]


def _tc_stream(pred, tgt):
    return pl.pallas_call(
        _tc_stream_kernel,
        grid=(_NBLK,),
        in_specs=[
            pl.BlockSpec((_B, 1), lambda i: (0, 0)),
            pl.BlockSpec((_B, _W), lambda i: (0, _TC0 + i)),
        ],
        out_specs=[
            pl.BlockSpec((_B, 1), lambda i: (0, 0)),
            pl.BlockSpec((_B, 1), lambda i: (0, 0)),
            pl.BlockSpec((_B, 1), lambda i: (0, 0)),
            pl.BlockSpec((_B, 1), lambda i: (0, 0)),
        ],
        out_shape=[jax.ShapeDtypeStruct((_B, 1), jnp.float32)] * 4,
        scratch_shapes=[pltpu.VMEM((_B, 1), jnp.float32)] * 4,
        compiler_params=pltpu.CompilerParams(
            dimension_semantics=("arbitrary",),
        ),
    )(tgt, pred)


def _sc_kernel(pred2d, tgt, m_hbm, se_hbm, rs_hbm, g_hbm,
               tgt_v, buf, macc, sacc, rsacc, m_v, se_v, rs_v, g_v,
               sem0, sem1):
    wid = lax.axis_index("s") * 2 + lax.axis_index("c")
    base = wid * _RPW
    lanes = lax.broadcasted_iota(jnp.int32, (16,), 0)

    pltpu.sync_copy(tgt.at[pl.ds(base, _RPW)], tgt_v)
    t0 = tgt_v[pl.ds(0, 16)]
    t1 = tgt_v[pl.ds(16, 16)]

    for r in range(_RPW):
        macc[r, :] = jnp.full((16,), _NEG_INF, jnp.float32)
        sacc[r, :] = jnp.zeros((16,), jnp.float32)
        rsacc[r, :] = jnp.zeros((16,), jnp.float32)

    def _start(ch, b):
        return pltpu.async_copy(
            pred2d.at[pl.ds(base, _RPW), pl.ds(ch * _CW, _CW)],
            buf.at[b],
            sem0 if b == 0 else sem1,
        )

    def _wait(b):
        pltpu.make_async_copy(
            pred2d.at[pl.ds(0, _RPW), pl.ds(0, _CW)],
            buf.at[b],
            sem0 if b == 0 else sem1,
        ).wait()

    neg16 = jnp.full((16,), _NEG_INF, jnp.float32)
    zero16 = jnp.zeros((16,), jnp.float32)

    def _consume(b, ch, g0, g1):
        for r in range(_RPW):
            def _p1(j, carry, b=b, r=r):
                bmv, rv = carry
                x = buf.at[b, r][pl.ds(j, 16)]
                return jnp.maximum(bmv, x), rv + x

            bmv, rv = plsc.parallel_loop(
                0, _CW, step=16, unroll=8, carry=(neg16, rsacc[r, :])
            )(_p1)
            mv = macc[r, :]
            nm = jnp.maximum(mv, bmv)
            scale = jnp.exp(mv - nm)

            def _p2(j, sv, b=b, r=r, nm=nm):
                x = buf.at[b, r][pl.ds(j, 16)]
                return sv + jnp.exp(x - nm)

            sv0 = plsc.parallel_loop(
                0, _CW, step=16, unroll=8, carry=zero16
            )(_p2)
            sacc[r, :] = sacc[r, :] * scale + sv0
            macc[r, :] = nm
            rsacc[r, :] = rv

        lo = ch * _CW
        out = []
        for k, tk in ((0, g0), (1, g1)):
            c_in = (t0 if k == 0 else t1) - lo
            hit = (c_in >= 0) & (c_in < _CW)
            c_cl = jnp.clip(c_in, 0, _CW - 1)
            val = plsc.load_gather(buf.at[b], [k * 16 + lanes, c_cl])
            out.append(jnp.where(hit, val, tk))
        return out[0], out[1]

    _start(0, 0)
    _start(1, 1)

    def _pair(i, carry):
        g0, g1 = carry
        ch0 = 2 * i
        _wait(0)
        g0, g1 = _consume(0, ch0, g0, g1)

        @pl.when(ch0 + 2 < _NCH)
        def _s0():
            _start(ch0 + 2, 0)

        _wait(1)
        g0, g1 = _consume(1, ch0 + 1, g0, g1)

        @pl.when(ch0 + 3 < _NCH)
        def _s1():
            _start(ch0 + 3, 1)

        return g0, g1

    g0, g1 = lax.fori_loop(0, _NCH // 2, _pair, (zero16, zero16))
    g_v[pl.ds(0, 16)] = g0
    g_v[pl.ds(16, 16)] = g1

    mrow_vec = zero16
    serow_vec = zero16
    rsrow_vec = zero16
    for r in range(_RPW):
        mv = macc[r, :]
        sv = sacc[r, :]
        rv = rsacc[r, :]
        mrow = jnp.max(mv)
        serow = jnp.sum(sv * jnp.exp(mv - jnp.broadcast_to(mrow, (16,))))
        rsrow = jnp.sum(rv)
        hot = lanes == (r % 16)
        mrow_vec = jnp.where(hot, jnp.broadcast_to(mrow, (16,)), mrow_vec)
        serow_vec = jnp.where(hot, jnp.broadcast_to(serow, (16,)), serow_vec)
        rsrow_vec = jnp.where(hot, jnp.broadcast_to(rsrow, (16,)), rsrow_vec)
        if r % 16 == 15:
            o = (r // 16) * 16
            m_v[pl.ds(o, 16)] = mrow_vec
            se_v[pl.ds(o, 16)] = serow_vec
            rs_v[pl.ds(o, 16)] = rsrow_vec
    pltpu.sync_copy(m_v, m_hbm.at[pl.ds(base, _RPW)])
    pltpu.sync_copy(se_v, se_hbm.at[pl.ds(base, _RPW)])
    pltpu.sync_copy(rs_v, rs_hbm.at[pl.ds(base, _RPW)])
    pltpu.sync_copy(g_v, g_hbm.at[pl.ds(base, _RPW)])


def _sc_stream(pred, tgt):
    mesh = plsc.VectorSubcoreMesh(core_axis_name="c", subcore_axis_name="s")
    f = functools.partial(
        pl.kernel,
        out_type=[jax.ShapeDtypeStruct((_B,), jnp.float32)] * 4,
        mesh=mesh,
        scratch_types=[
            pltpu.VMEM((_RPW,), jnp.int32),
            pltpu.VMEM((2, _RPW, _CW), jnp.float32),
            pltpu.VMEM((_RPW, 16), jnp.float32),
            pltpu.VMEM((_RPW, 16), jnp.float32),
            pltpu.VMEM((_RPW, 16), jnp.float32),
            pltpu.VMEM((_RPW,), jnp.float32),
            pltpu.VMEM((_RPW,), jnp.float32),
            pltpu.VMEM((_RPW,), jnp.float32),
            pltpu.VMEM((_RPW,), jnp.float32),
            pltpu.SemaphoreType.DMA,
            pltpu.SemaphoreType.DMA,
        ],
        compiler_params=pltpu.CompilerParams(
            use_tc_tiling_on_sc=False, needs_layout_passes=False
        ),
    )(_sc_kernel)
    return f(pred, tgt)


def _merge_kernel(m1_ref, se1_ref, rs1_ref, g1_ref, m2_ref, se2_ref,
                  rs2_ref, g2_ref, out_ref):
    m1 = m1_ref[...]
    m2 = m2_ref[...]
    nm = jnp.maximum(m1, m2)
    se = se1_ref[...] * jnp.exp(m1 - nm) + se2_ref[...] * jnp.exp(m2 - nm)
    z = nm + jnp.log(se)
    zsum = jnp.sum(z)
    rssum = jnp.sum(rs1_ref[...]) + jnp.sum(rs2_ref[...])
    gsum = jnp.sum(g1_ref[...]) + jnp.sum(g2_ref[...])
    kc = _SMOOTHING * math.log(_S) + _CONF * math.log(_CONF)
    total = (
        _B * kc
        - _S * (rssum - _C * zsum)
        - (_CONF - _S) * (gsum - zsum)
    )
    out_ref[0, 0] = total / (_B * _C)


def _merge(m1, se1, rs1, g1, m2, se2, rs2, g2):
    return pl.pallas_call(
        _merge_kernel,
        in_specs=[pl.BlockSpec((_B, 1), lambda: (0, 0))] * 8,
        out_specs=pl.BlockSpec((1, 1), lambda: (0, 0), memory_space=pltpu.SMEM),
        out_shape=jax.ShapeDtypeStruct((1, 1), jnp.float32),
    )(m1, se1, rs1, g1, m2, se2, rs2, g2)


def kernel(pred, target):
    tgt = target.astype(jnp.int32)
    m2, se2, rs2, g2 = _sc_stream(pred, tgt)
    m1, se1, rs1, g1 = _tc_stream(pred, tgt.reshape(_B, 1))
    out = _merge(
        m1, se1, rs1, g1,
        m2.reshape(_B, 1), se2.reshape(_B, 1), rs2.reshape(_B, 1),
        g2.reshape(_B, 1),
    )
    return out[0, 0]


# speedup vs baseline: 1.4407x; 1.0739x over previous
import functools
import math

import jax
import jax.numpy as jnp
from jax import lax
from jax.experimental import pallas as pl
from jax.experimental.pallas import tpu as pltpu
from jax.experimental.pallas import tpu_sc as plsc

_C = 100000
_B = 1024
_SMOOTHING = 0.1
_CONF = 1.0 - _SMOOTHING
_S = _SMOOTHING / (_C - 1)

_CW = 1024
_NCH = 66
_SW = _NCH * _CW
_W = 2048
_TC0 = _SW // _W
_NBLK = (_C - _SW + _W - 1) // _W

_NW = 32
_RPW = _B // _NW
_NEG_INF = float("-inf")


def _tc_stream_kernel(tgt_ref, x_ref, m_out, se_out, rs_out, g_out,
                      m_ref, se_ref, rs_ref, g_ref):
    i = pl.program_id(0)

    @pl.when(i == 0)
    def _init():
        m_ref[...] = jnp.full_like(m_ref, _NEG_INF)
        se_ref[...] = jnp.zeros_like(se_ref)
        rs_ref[...] = jnp.zeros_like(rs_ref)
        g_ref[...] = jnp.zeros_like(g_ref)

    x = x_ref[...]
    m = m_ref[...]
    col = jax.lax.broadcasted_iota(jnp.int32, x.shape, 1) + (_TC0 + i) * _W
    g_ref[...] += jnp.sum(
        jnp.where(col == tgt_ref[...], x, 0.0), axis=1, keepdims=True
    )

    @pl.when(i < _NBLK - 1)
    def _main():
        bm = jnp.max(x, axis=1, keepdims=True)
        nm = jnp.maximum(m, bm)
        se_ref[...] = se_ref[...] * jnp.exp(m - nm) + jnp.sum(
            jnp.exp(x - nm), axis=1, keepdims=True
        )
        m_ref[...] = nm
        rs_ref[...] += jnp.sum(x, axis=1, keepdims=True)

    @pl.when(i == _NBLK - 1)
    def _last():
        valid = col < _C
        xm = jnp.where(valid, x, _NEG_INF)
        bm = jnp.max(xm, axis=1, keepdims=True)
        nm = jnp.maximum(m, bm)
        se_ref[...] = se_ref[...] * jnp.exp(m - nm) + jnp.sum(
            jnp.exp(xm - nm), axis=1, keepdims=True
        )
        rs_ref[...] += jnp.sum(jnp.where(valid, x, 0.0), axis=1, keepdims=True)
        m_out[...] = nm
        se_out[...] = se_ref[...]
        rs_out[...] = rs_ref[...]
        g_out[...] = g_ref[...]


def _tc_stream(pred, tgt):
    return pl.pallas_call(
        _tc_stream_kernel,
        grid=(_NBLK,),
        in_specs=[
            pl.BlockSpec((_B, 1), lambda i: (0, 0)),
            pl.BlockSpec((_B, _W), lambda i: (0, _TC0 + i)),
        ],
        out_specs=[
            pl.BlockSpec((_B, 1), lambda i: (0, 0)),
            pl.BlockSpec((_B, 1), lambda i: (0, 0)),
            pl.BlockSpec((_B, 1), lambda i: (0, 0)),
            pl.BlockSpec((_B, 1), lambda i: (0, 0)),
        ],
        out_shape=[jax.ShapeDtypeStruct((_B, 1), jnp.float32)] * 4,
        scratch_shapes=[pltpu.VMEM((_B, 1), jnp.float32)] * 4,
        compiler_params=pltpu.CompilerParams(
            dimension_semantics=("arbitrary",),
        ),
    )(tgt, pred)


def _sc_kernel(pred2d, tgt, m_hbm, se_hbm, rs_hbm, g_hbm,
               tgt_v, buf, macc, sacc, rsacc, m_v, se_v, rs_v, g_v,
               sem0, sem1):
    wid = lax.axis_index("s") * 2 + lax.axis_index("c")
    base = wid * _RPW
    lanes = lax.broadcasted_iota(jnp.int32, (16,), 0)

    pltpu.sync_copy(tgt.at[pl.ds(base, _RPW)], tgt_v)
    t0 = tgt_v[pl.ds(0, 16)]
    t1 = tgt_v[pl.ds(16, 16)]

    neg16 = jnp.full((16,), _NEG_INF, jnp.float32)
    zero16 = jnp.zeros((16,), jnp.float32)

    def _initloop(i, _):
        macc[pl.ds(i, 16)] = neg16
        sacc[pl.ds(i, 16)] = zero16
        rsacc[pl.ds(i, 16)] = zero16
        return _

    plsc.parallel_loop(0, _RPW * 16, step=16, carry=jnp.int32(0))(_initloop)

    def _start(ch, b):
        return pltpu.async_copy(
            pred2d.at[pl.ds(base, _RPW), pl.ds(ch * _CW, _CW)],
            buf.at[b],
            sem0 if b == 0 else sem1,
        )

    def _wait(b):
        pltpu.make_async_copy(
            pred2d.at[pl.ds(0, _RPW), pl.ds(0, _CW)],
            buf.at[b],
            sem0 if b == 0 else sem1,
        ).wait()

    def _consume(b, ch, g0, g1):
        def _row(r, carry):
            g0, g1 = carry
            aidx = r * 16 + lanes
            rfull = jnp.broadcast_to(r, (16,))

            def _p1(j, c, b=b, rfull=rfull):
                bmv, rv = c
                x = plsc.load_gather(buf.at[b], [rfull, j + lanes])
                return jnp.maximum(bmv, x), rv + x

            bmv, rv0 = plsc.parallel_loop(
                0, _CW, step=16, unroll=8, carry=(neg16, zero16)
            )(_p1)
            mv = plsc.load_gather(macc, [aidx])
            sv = plsc.load_gather(sacc, [aidx])
            rv = plsc.load_gather(rsacc, [aidx])
            nm = jnp.maximum(mv, bmv)
            scale = jnp.exp(mv - nm)

            def _p2(j, sv0, b=b, rfull=rfull, nm=nm):
                x = plsc.load_gather(buf.at[b], [rfull, j + lanes])
                return sv0 + jnp.exp(x - nm)

            sv0 = plsc.parallel_loop(
                0, _CW, step=16, unroll=8, carry=zero16
            )(_p2)
            plsc.store_scatter(macc, [aidx], nm)
            plsc.store_scatter(sacc, [aidx], sv * scale + sv0)
            plsc.store_scatter(rsacc, [aidx], rv + rv0)
            return g0, g1

        g0, g1 = lax.fori_loop(0, _RPW, _row, (g0, g1))

        lo = ch * _CW
        out = []
        for k, tk in ((0, g0), (1, g1)):
            c_in = (t0 if k == 0 else t1) - lo
            hit = (c_in >= 0) & (c_in < _CW)
            c_cl = jnp.clip(c_in, 0, _CW - 1)
            val = plsc.load_gather(buf.at[b], [k * 16 + lanes, c_cl])
            out.append(jnp.where(hit, val, tk))
        return out[0], out[1]

    _start(0, 0)
    _start(1, 1)

    def _pair(i, carry):
        g0, g1 = carry
        ch0 = 2 * i
        _wait(0)
        g0, g1 = _consume(0, ch0, g0, g1)

        @pl.when(ch0 + 2 < _NCH)
        def _s0():
            _start(ch0 + 2, 0)

        _wait(1)
        g0, g1 = _consume(1, ch0 + 1, g0, g1)

        @pl.when(ch0 + 3 < _NCH)
        def _s1():
            _start(ch0 + 3, 1)

        return g0, g1

    g0, g1 = lax.fori_loop(0, _NCH // 2, _pair, (zero16, zero16))
    g_v[pl.ds(0, 16)] = g0
    g_v[pl.ds(16, 16)] = g1

    lane0 = lanes == 0

    def _ep(r, _):
        aidx = r * 16 + lanes
        mv = plsc.load_gather(macc, [aidx])
        sv = plsc.load_gather(sacc, [aidx])
        rv = plsc.load_gather(rsacc, [aidx])
        mrow = jnp.max(mv)
        serow = jnp.sum(sv * jnp.exp(mv - jnp.broadcast_to(mrow, (16,))))
        rsrow = jnp.sum(rv)
        ridx = jnp.broadcast_to(r, (16,))
        plsc.store_scatter(m_v, [ridx], jnp.broadcast_to(mrow, (16,)), mask=lane0)
        plsc.store_scatter(se_v, [ridx], jnp.broadcast_to(serow, (16,)), mask=lane0)
        plsc.store_scatter(rs_v, [ridx], jnp.broadcast_to(rsrow, (16,)), mask=lane0)
        return _

    lax.fori_loop(0, _RPW, _ep, 0)
    pltpu.sync_copy(m_v, m_hbm.at[pl.ds(base, _RPW)])
    pltpu.sync_copy(se_v, se_hbm.at[pl.ds(base, _RPW)])
    pltpu.sync_copy(rs_v, rs_hbm.at[pl.ds(base, _RPW)])
    pltpu.sync_copy(g_v, g_hbm.at[pl.ds(base, _RPW)])


def _sc_stream(pred, tgt):
    mesh = plsc.VectorSubcoreMesh(core_axis_name="c", subcore_axis_name="s")
    f = functools.partial(
        pl.kernel,
        out_type=[jax.ShapeDtypeStruct((_B,), jnp.float32)] * 4,
        mesh=mesh,
        scratch_types=[
            pltpu.VMEM((_RPW,), jnp.int32),
            pltpu.VMEM((2, _RPW, _CW), jnp.float32),
            pltpu.VMEM((_RPW * 16,), jnp.float32),
            pltpu.VMEM((_RPW * 16,), jnp.float32),
            pltpu.VMEM((_RPW * 16,), jnp.float32),
            pltpu.VMEM((_RPW,), jnp.float32),
            pltpu.VMEM((_RPW,), jnp.float32),
            pltpu.VMEM((_RPW,), jnp.float32),
            pltpu.VMEM((_RPW,), jnp.float32),
            pltpu.SemaphoreType.DMA,
            pltpu.SemaphoreType.DMA,
        ],
        compiler_params=pltpu.CompilerParams(
            use_tc_tiling_on_sc=False, needs_layout_passes=False
        ),
    )(_sc_kernel)
    return f(pred, tgt)


def _merge_kernel(m1_ref, se1_ref, rs1_ref, g1_ref, m2_ref, se2_ref,
                  rs2_ref, g2_ref, out_ref):
    m1 = m1_ref[...]
    m2 = m2_ref[...]
    nm = jnp.maximum(m1, m2)
    se = se1_ref[...] * jnp.exp(m1 - nm) + se2_ref[...] * jnp.exp(m2 - nm)
    z = nm + jnp.log(se)
    zsum = jnp.sum(z)
    rssum = jnp.sum(rs1_ref[...]) + jnp.sum(rs2_ref[...])
    gsum = jnp.sum(g1_ref[...]) + jnp.sum(g2_ref[...])
    kc = _SMOOTHING * math.log(_S) + _CONF * math.log(_CONF)
    total = (
        _B * kc
        - _S * (rssum - _C * zsum)
        - (_CONF - _S) * (gsum - zsum)
    )
    out_ref[0, 0] = total / (_B * _C)


def _merge(m1, se1, rs1, g1, m2, se2, rs2, g2):
    return pl.pallas_call(
        _merge_kernel,
        in_specs=[pl.BlockSpec((_B, 1), lambda: (0, 0))] * 8,
        out_specs=pl.BlockSpec((1, 1), lambda: (0, 0), memory_space=pltpu.SMEM),
        out_shape=jax.ShapeDtypeStruct((1, 1), jnp.float32),
    )(m1, se1, rs1, g1, m2, se2, rs2, g2)


def kernel(pred, target):
    tgt = target.astype(jnp.int32)
    m2, se2, rs2, g2 = _sc_stream(pred, tgt)
    m1, se1, rs1, g1 = _tc_stream(pred, tgt.reshape(_B, 1))
    out = _merge(
        m1, se1, rs1, g1,
        m2.reshape(_B, 1), se2.reshape(_B, 1), rs2.reshape(_B, 1),
        g2.reshape(_B, 1),
    )
    return out[0, 0]
